# baseline (speedup 1.0000x reference)
import math
import jax
import jax.numpy as jnp
from jax.experimental import pallas as pl
from jax.experimental.pallas import tpu as pltpu

_B, _L, _IN, _D, _H, _LAYERS = 1, 2048, 256, 1024, 16, 2
_DH = _D // _H
_FF = 4 * _D
_SCALE = 1.0 / math.sqrt(_DH)

_LB = 256
_LQB = 256
_FB = 1024

_BF = jnp.bfloat16
_F32 = jnp.float32


def _pos_enc():
    pos = jnp.arange(_L, dtype=_F32)[:, None]
    div = jnp.exp(jnp.arange(0, _D, 2, dtype=_F32) * (-math.log(10000.0) / _D))
    pe = jnp.zeros((_L, _D), _F32)
    pe = pe.at[:, 0::2].set(jnp.sin(pos * div))
    pe = pe.at[:, 1::2].set(jnp.cos(pos * div))
    return pe


def _ln_rows(x, g, b):
    m = jnp.mean(x, axis=-1, keepdims=True)
    v = jnp.mean((x - m) ** 2, axis=-1, keepdims=True)
    return (x - m) * jax.lax.rsqrt(v + 1e-5) * g + b


def _bdot(a, b):
    return jnp.dot(a.astype(_BF), b, preferred_element_type=_F32)


def _inproj_body(x_ref, w_ref, b_ref, pe_ref, t_ref, o_ref):
    o_ref[...] = (_bdot(x_ref[...], w_ref[...])
                  + b_ref[...] + pe_ref[...] + t_ref[...])


def _attn_layer_body(hfull_ref, wq_ref, bq_ref, wk_ref, bk_ref,
                     wv_ref, bv_ref, wo_ref, bo_ref, g_ref, be_ref,
                     o_ref, k_sc, v_sc):
    i = pl.program_id(0)

    @pl.when(i == 0)
    def _():
        hb = hfull_ref[...].astype(_BF)
        k_sc[...] = (jnp.dot(hb, wk_ref[...], preferred_element_type=_F32)
                     + bk_ref[...]).astype(_BF)
        v_sc[...] = (jnp.dot(hb, wv_ref[...], preferred_element_type=_F32)
                     + bv_ref[...]).astype(_BF)

    hrows = hfull_ref[pl.ds(i * _LQB, _LQB), :]
    q = (jnp.dot(hrows.astype(_BF), wq_ref[...],
                 preferred_element_type=_F32) + bq_ref[...]).astype(_BF)
    cols = []
    for hd in range(_H):
        sl = slice(hd * _DH, (hd + 1) * _DH)
        s = jax.lax.dot_general(q[:, sl], k_sc[:, sl],
                                (((1,), (1,)), ((), ())),
                                preferred_element_type=_F32) * _SCALE
        m = jnp.max(s, axis=-1, keepdims=True)
        p = jnp.exp(s - m)
        denom = jnp.sum(p, axis=-1, keepdims=True)
        ctx = jnp.dot(p.astype(_BF), v_sc[:, sl],
                      preferred_element_type=_F32)
        cols.append((ctx / denom).astype(_BF))
    ctx_all = jnp.concatenate(cols, axis=1)
    attn = jnp.dot(ctx_all, wo_ref[...],
                   preferred_element_type=_F32) + bo_ref[...]
    o_ref[...] = _ln_rows(hrows + attn, g_ref[...], be_ref[...])


def _ffn_body(h_ref, w1_ref, b1_ref, w2_ref, b2_ref, g_ref, be_ref,
              o_ref, acc_ref):
    j = pl.program_id(0)
    u = _bdot(h_ref[...], w1_ref[...]) + b1_ref[...]
    u = 0.5 * u * (1.0 + jax.lax.erf(u * (1.0 / math.sqrt(2.0))))
    p = jnp.dot(u.astype(_BF), w2_ref[...], preferred_element_type=_F32)

    @pl.when(j == 0)
    def _():
        acc_ref[...] = p

    @pl.when(j > 0)
    def _():
        acc_ref[...] += p

    @pl.when(j == (_FF // _FB) - 1)
    def _():
        r = h_ref[...] + acc_ref[...] + b2_ref[...]
        o_ref[...] = _ln_rows(r, g_ref[...], be_ref[...])


def _outproj_body(h_ref, w_ref, b_ref, o_ref):
    o_ref[...] = _bdot(h_ref[...], w_ref[...]) + b_ref[...]


def _row_spec(rb, cols):
    return pl.BlockSpec((rb, cols), lambda i: (i, 0))


def _full_spec(shape):
    return pl.BlockSpec(shape, lambda i: tuple(0 for _ in shape))


def _inproj(x, win, b_in, pe, t):
    return pl.pallas_call(
        _inproj_body,
        grid=(_L // _LB,),
        in_specs=[_row_spec(_LB, _IN), _full_spec((_IN, _D)),
                  _full_spec((1, _D)), _row_spec(_LB, _D), _full_spec((1, _D))],
        out_specs=_row_spec(_LB, _D),
        out_shape=jax.ShapeDtypeStruct((_L, _D), _F32),
    )(x, win, b_in, pe, t)


def _attn_layer(h, wq, bq, wk, bk, wv, bv, wo, bo, g, be):
    return pl.pallas_call(
        _attn_layer_body,
        grid=(_L // _LQB,),
        in_specs=[_full_spec((_L, _D)),
                  _full_spec((_D, _D)), _full_spec((1, _D)),
                  _full_spec((_D, _D)), _full_spec((1, _D)),
                  _full_spec((_D, _D)), _full_spec((1, _D)),
                  _full_spec((_D, _D)), _full_spec((1, _D)),
                  _full_spec((1, _D)), _full_spec((1, _D))],
        out_specs=_row_spec(_LQB, _D),
        out_shape=jax.ShapeDtypeStruct((_L, _D), _F32),
        scratch_shapes=[pltpu.VMEM((_L, _D), _BF),
                        pltpu.VMEM((_L, _D), _BF)],
    )(h, wq, bq, wk, bk, wv, bv, wo, bo, g, be)


def _ffn(h, w1, b1, w2, b2, g, be):
    nf = _FF // _FB
    return pl.pallas_call(
        _ffn_body,
        grid=(nf,),
        in_specs=[_full_spec((_L, _D)),
                  pl.BlockSpec((_D, _FB), lambda j: (0, j)),
                  pl.BlockSpec((1, _FB), lambda j: (0, j)),
                  pl.BlockSpec((_FB, _D), lambda j: (j, 0)),
                  _full_spec((1, _D)), _full_spec((1, _D)), _full_spec((1, _D))],
        out_specs=_full_spec((_L, _D)),
        out_shape=jax.ShapeDtypeStruct((_L, _D), _F32),
        scratch_shapes=[pltpu.VMEM((_L, _D), _F32)],
    )(h, w1, b1, w2, b2, g, be)


def _outproj(h, wout, bout):
    return pl.pallas_call(
        _outproj_body,
        grid=(_L // _LB,),
        in_specs=[_row_spec(_LB, _D), _full_spec((_D, _IN)),
                  _full_spec((1, _IN))],
        out_specs=_row_spec(_LB, _IN),
        out_shape=jax.ShapeDtypeStruct((_L, _IN), _F32),
    )(h, wout, bout)


def kernel(x_t, t_embed, Win, b_in, Wq, bq, Wk, bk, Wv, bv, Wo, bo,
           W1, b1, W2, b2, g1, be1, g2, be2, Wout, bout):
    x = x_t.reshape(_L, _IN)
    pe = _pos_enc()
    h = _inproj(x, Win.astype(_BF), b_in.reshape(1, _D), pe,
                t_embed.reshape(1, _D))
    Wqb, Wkb, Wvb, Wob = (w.astype(_BF) for w in (Wq, Wk, Wv, Wo))
    W1b, W2b = W1.astype(_BF), W2.astype(_BF)
    for i in range(_LAYERS):
        h = _attn_layer(h, Wqb[i], bq[i].reshape(1, _D),
                        Wkb[i], bk[i].reshape(1, _D),
                        Wvb[i], bv[i].reshape(1, _D),
                        Wob[i], bo[i].reshape(1, _D),
                        g1[i].reshape(1, _D), be1[i].reshape(1, _D))
        h = _ffn(h, W1b[i], b1[i].reshape(1, _FF), W2b[i],
                 b2[i].reshape(1, _D), g2[i].reshape(1, _D),
                 be2[i].reshape(1, _D))
    out = _outproj(h, Wout.astype(_BF), bout.reshape(1, _IN))
    return out.reshape(_B, _L, _IN)


# speedup vs baseline: 1.0870x; 1.0870x over previous
import math
import jax
import jax.numpy as jnp
from jax.experimental import pallas as pl
from jax.experimental.pallas import tpu as pltpu

_B, _L, _IN, _D, _H, _LAYERS = 1, 2048, 256, 1024, 16, 2
_DH = _D // _H
_FF = 4 * _D
_SCALE = 1.0 / math.sqrt(_DH)

_LB = 256
_LQB = 256
_FB = 1024

_BF = jnp.bfloat16
_F32 = jnp.float32


def _pos_enc():
    pos = jnp.arange(_L, dtype=_F32)[:, None]
    div = jnp.exp(jnp.arange(0, _D, 2, dtype=_F32) * (-math.log(10000.0) / _D))
    pe = jnp.zeros((_L, _D), _F32)
    pe = pe.at[:, 0::2].set(jnp.sin(pos * div))
    pe = pe.at[:, 1::2].set(jnp.cos(pos * div))
    return pe


def _ln_rows(x, g, b):
    m = jnp.mean(x, axis=-1, keepdims=True)
    v = jnp.mean((x - m) ** 2, axis=-1, keepdims=True)
    return (x - m) * jax.lax.rsqrt(v + 1e-5) * g + b


def _bdot(a, b):
    return jnp.dot(a.astype(_BF), b, preferred_element_type=_F32)


def _inproj_body(x_ref, w_ref, b_ref, pe_ref, t_ref, o_ref):
    o_ref[...] = (_bdot(x_ref[...], w_ref[...])
                  + b_ref[...] + pe_ref[...] + t_ref[...])


def _qkv_body(h_ref, wq_ref, bq_ref, wk_ref, bk_ref, wv_ref, bv_ref,
              q_ref, k_ref, v_ref):
    h = h_ref[...].astype(_BF)
    q_ref[...] = (jnp.dot(h, wq_ref[...], preferred_element_type=_F32)
                  + bq_ref[...]).astype(_BF)
    k_ref[...] = (jnp.dot(h, wk_ref[...], preferred_element_type=_F32)
                  + bk_ref[...]).astype(_BF)
    v_ref[...] = (jnp.dot(h, wv_ref[...], preferred_element_type=_F32)
                  + bv_ref[...]).astype(_BF)


def _attn_oproj_ln_body(q_ref, k_ref, v_ref, h_ref, wo_ref, bo_ref,
                        g_ref, be_ref, o_ref):
    cols = []
    for hd in range(_H):
        sl = slice(hd * _DH, (hd + 1) * _DH)
        s = jax.lax.dot_general(q_ref[:, sl], k_ref[:, sl],
                                (((1,), (1,)), ((), ())),
                                preferred_element_type=_F32)
        sb = (s * _SCALE).astype(_BF)
        m = jnp.max(sb, axis=-1, keepdims=True)
        p = jnp.exp(sb - m)
        cd = jnp.dot(p, v_ref[:, hd * 128:(hd + 1) * 128],
                     preferred_element_type=_F32)
        ctx = cd[:, :_DH]
        denom = cd[:, _DH:_DH + 1]
        cols.append((ctx / denom).astype(_BF))
    ctx_all = jnp.concatenate(cols, axis=1)
    attn = jnp.dot(ctx_all, wo_ref[...],
                   preferred_element_type=_F32) + bo_ref[...]
    o_ref[...] = _ln_rows(h_ref[...] + attn, g_ref[...], be_ref[...])


def _ffn_body(h_ref, w1_ref, b1_ref, w2_ref, b2_ref, g_ref, be_ref,
              o_ref, acc_ref):
    j = pl.program_id(0)
    u = _bdot(h_ref[...], w1_ref[...]) + b1_ref[...]
    u = 0.5 * u * (1.0 + jax.lax.erf(u * (1.0 / math.sqrt(2.0))))
    p = jnp.dot(u.astype(_BF), w2_ref[...], preferred_element_type=_F32)

    @pl.when(j == 0)
    def _():
        acc_ref[...] = p

    @pl.when(j > 0)
    def _():
        acc_ref[...] += p

    @pl.when(j == (_FF // _FB) - 1)
    def _():
        r = h_ref[...] + acc_ref[...] + b2_ref[...]
        o_ref[...] = _ln_rows(r, g_ref[...], be_ref[...])


def _outproj_body(h_ref, w_ref, b_ref, o_ref):
    o_ref[...] = _bdot(h_ref[...], w_ref[...]) + b_ref[...]


def _row_spec(rb, cols):
    return pl.BlockSpec((rb, cols), lambda i: (i, 0))


def _full_spec(shape):
    return pl.BlockSpec(shape, lambda i: tuple(0 for _ in shape))


def _inproj(x, win, b_in, pe, t):
    return pl.pallas_call(
        _inproj_body,
        grid=(_L // _LB,),
        in_specs=[_row_spec(_LB, _IN), _full_spec((_IN, _D)),
                  _full_spec((1, _D)), _row_spec(_LB, _D), _full_spec((1, _D))],
        out_specs=_row_spec(_LB, _D),
        out_shape=jax.ShapeDtypeStruct((_L, _D), _F32),
    )(x, win, b_in, pe, t)


_VW = _H * 128


def _qkv(h, wq, bq, wk, bk, wv, bv):
    s = jax.ShapeDtypeStruct((_L, _D), _BF)
    sv = jax.ShapeDtypeStruct((_L, _VW), _BF)
    return pl.pallas_call(
        _qkv_body,
        grid=(_L // _LB,),
        in_specs=[_row_spec(_LB, _D),
                  _full_spec((_D, _D)), _full_spec((1, _D)),
                  _full_spec((_D, _D)), _full_spec((1, _D)),
                  _full_spec((_D, _VW)), _full_spec((1, _VW))],
        out_specs=[_row_spec(_LB, _D), _row_spec(_LB, _D),
                   _row_spec(_LB, _VW)],
        out_shape=[s, s, sv],
    )(h, wq, bq, wk, bk, wv, bv)


def _attn_oproj_ln(q, k, v, h, wo, bo, g, be):
    return pl.pallas_call(
        _attn_oproj_ln_body,
        grid=(_L // _LQB,),
        in_specs=[_row_spec(_LQB, _D), _full_spec((_L, _D)),
                  _full_spec((_L, _VW)), _row_spec(_LQB, _D),
                  _full_spec((_D, _D)), _full_spec((1, _D)),
                  _full_spec((1, _D)), _full_spec((1, _D))],
        out_specs=_row_spec(_LQB, _D),
        out_shape=jax.ShapeDtypeStruct((_L, _D), _F32),
    )(q, k, v, h, wo, bo, g, be)


def _ffn(h, w1, b1, w2, b2, g, be):
    nf = _FF // _FB
    return pl.pallas_call(
        _ffn_body,
        grid=(nf,),
        in_specs=[_full_spec((_L, _D)),
                  pl.BlockSpec((_D, _FB), lambda j: (0, j)),
                  pl.BlockSpec((1, _FB), lambda j: (0, j)),
                  pl.BlockSpec((_FB, _D), lambda j: (j, 0)),
                  _full_spec((1, _D)), _full_spec((1, _D)), _full_spec((1, _D))],
        out_specs=_full_spec((_L, _D)),
        out_shape=jax.ShapeDtypeStruct((_L, _D), _F32),
        scratch_shapes=[pltpu.VMEM((_L, _D), _F32)],
    )(h, w1, b1, w2, b2, g, be)


def _outproj(h, wout, bout):
    return pl.pallas_call(
        _outproj_body,
        grid=(_L // _LB,),
        in_specs=[_row_spec(_LB, _D), _full_spec((_D, _IN)),
                  _full_spec((1, _IN))],
        out_specs=_row_spec(_LB, _IN),
        out_shape=jax.ShapeDtypeStruct((_L, _IN), _F32),
    )(h, wout, bout)


def kernel(x_t, t_embed, Win, b_in, Wq, bq, Wk, bk, Wv, bv, Wo, bo,
           W1, b1, W2, b2, g1, be1, g2, be2, Wout, bout):
    x = x_t.reshape(_L, _IN)
    pe = _pos_enc()
    h = _inproj(x, Win.astype(_BF), b_in.reshape(1, _D), pe,
                t_embed.reshape(1, _D))
    Wqb, Wkb, Wob = (w.astype(_BF) for w in (Wq, Wk, Wo))
    W1b, W2b = W1.astype(_BF), W2.astype(_BF)
    Wv_aug = jnp.pad(Wv.reshape(_LAYERS, _D, _H, _DH),
                     ((0, 0), (0, 0), (0, 0), (0, 128 - _DH))
                     ).reshape(_LAYERS, _D, _VW).astype(_BF)
    bv_aug = jnp.concatenate(
        [bv.reshape(_LAYERS, _H, _DH),
         jnp.ones((_LAYERS, _H, 1), _F32),
         jnp.zeros((_LAYERS, _H, 128 - _DH - 1), _F32)],
        axis=-1).reshape(_LAYERS, _VW)
    for i in range(_LAYERS):
        q, k, v = _qkv(h, Wqb[i], bq[i].reshape(1, _D), Wkb[i],
                       bk[i].reshape(1, _D), Wv_aug[i],
                       bv_aug[i].reshape(1, _VW))
        h = _attn_oproj_ln(q, k, v, h, Wob[i], bo[i].reshape(1, _D),
                           g1[i].reshape(1, _D), be1[i].reshape(1, _D))
        h = _ffn(h, W1b[i], b1[i].reshape(1, _FF), W2b[i],
                 b2[i].reshape(1, _D), g2[i].reshape(1, _D),
                 be2[i].reshape(1, _D))
    out = _outproj(h, Wout.astype(_BF), bout.reshape(1, _IN))
    return out.reshape(_B, _L, _IN)


# speedup vs baseline: 1.2319x; 1.1333x over previous
import math
import jax
import jax.numpy as jnp
from jax.experimental import pallas as pl
from jax.experimental.pallas import tpu as pltpu

_B, _L, _IN, _D, _H, _LAYERS = 1, 2048, 256, 1024, 16, 2
_DH = _D // _H
_FF = 4 * _D
_SCALE = 1.0 / math.sqrt(_DH)

_LB = 512
_LQB = 512
_VW = _H * 128

_BF = jnp.bfloat16
_F32 = jnp.float32


def _pos_enc():
    pos = jnp.arange(_L, dtype=_F32)[:, None]
    div = jnp.exp(jnp.arange(0, _D, 2, dtype=_F32) * (-math.log(10000.0) / _D))
    pe = jnp.zeros((_L, _D), _F32)
    pe = pe.at[:, 0::2].set(jnp.sin(pos * div))
    pe = pe.at[:, 1::2].set(jnp.cos(pos * div))
    return pe


def _ln_rows(x, g, b):
    m = jnp.mean(x, axis=-1, keepdims=True)
    v = jnp.mean((x - m) ** 2, axis=-1, keepdims=True)
    return (x - m) * jax.lax.rsqrt(v + 1e-5) * g + b


def _bdot(a, b):
    return jnp.dot(a.astype(_BF), b, preferred_element_type=_F32)


def _inproj_body(x_ref, w_ref, b_ref, pe_ref, t_ref, o_ref):
    o_ref[...] = (_bdot(x_ref[...], w_ref[...])
                  + b_ref[...] + pe_ref[...] + t_ref[...])


def _qkv_body(h_ref, wq_ref, bq_ref, wk_ref, bk_ref, wv_ref, bv_ref,
              q_ref, k_ref, v_ref):
    h = h_ref[...].astype(_BF)
    q_ref[...] = jnp.dot(h, wq_ref[...],
                         preferred_element_type=_F32).astype(_BF) + bq_ref[...]
    k_ref[...] = jnp.dot(h, wk_ref[...],
                         preferred_element_type=_F32).astype(_BF) + bk_ref[...]
    v_ref[...] = jnp.dot(h, wv_ref[...],
                         preferred_element_type=_F32).astype(_BF) + bv_ref[...]


def _mega_body(q_ref, k_ref, v_ref, h_ref, wo_ref, bo_ref, g1_ref, be1_ref,
               w1_ref, b1_ref, w2_ref, b2_ref, g2_ref, be2_ref, o_ref):
    cols = []
    for hd in range(_H):
        sl = slice(hd * _DH, (hd + 1) * _DH)
        sb = jax.lax.dot_general(q_ref[:, sl], k_ref[:, sl],
                                 (((1,), (1,)), ((), ())),
                                 preferred_element_type=_F32).astype(_BF)
        m = jnp.max(sb, axis=-1, keepdims=True)
        p = jnp.exp2(sb - m)
        cd = jnp.dot(p, v_ref[:, hd * 128:(hd + 1) * 128],
                     preferred_element_type=_F32)
        cols.append((cd[:, :_DH] / cd[:, _DH:_DH + 1]).astype(_BF))
    ctx_all = jnp.concatenate(cols, axis=1)
    attn = jnp.dot(ctx_all, wo_ref[...],
                   preferred_element_type=_F32) + bo_ref[...]
    h1 = _ln_rows(h_ref[...] + attn, g1_ref[...], be1_ref[...])
    u = jnp.dot(h1.astype(_BF), w1_ref[...],
                preferred_element_type=_F32).astype(_BF) + b1_ref[...]
    c = jnp.asarray(1.0 / math.sqrt(2.0), _BF)
    u = jnp.asarray(0.5, _BF) * u * (jnp.asarray(1.0, _BF)
                                     + jax.lax.erf(u * c))
    ff = jnp.dot(u, w2_ref[...], preferred_element_type=_F32) + b2_ref[...]
    o_ref[...] = _ln_rows(h1 + ff, g2_ref[...], be2_ref[...])


def _outproj_body(h_ref, w_ref, b_ref, o_ref):
    o_ref[...] = _bdot(h_ref[...], w_ref[...]) + b_ref[...]


def _row_spec(rb, cols):
    return pl.BlockSpec((rb, cols), lambda i: (i, 0))


def _full_spec(shape):
    return pl.BlockSpec(shape, lambda i: tuple(0 for _ in shape))


def _inproj(x, win, b_in, pe, t):
    return pl.pallas_call(
        _inproj_body,
        grid=(_L // _LB,),
        in_specs=[_row_spec(_LB, _IN), _full_spec((_IN, _D)),
                  _full_spec((1, _D)), _row_spec(_LB, _D), _full_spec((1, _D))],
        out_specs=_row_spec(_LB, _D),
        out_shape=jax.ShapeDtypeStruct((_L, _D), _F32),
    )(x, win, b_in, pe, t)


def _qkv(h, wq, bq, wk, bk, wv, bv):
    s = jax.ShapeDtypeStruct((_L, _D), _BF)
    sv = jax.ShapeDtypeStruct((_L, _VW), _BF)
    return pl.pallas_call(
        _qkv_body,
        grid=(_L // _LB,),
        in_specs=[_row_spec(_LB, _D),
                  _full_spec((_D, _D)), _full_spec((1, _D)),
                  _full_spec((_D, _D)), _full_spec((1, _D)),
                  _full_spec((_D, _VW)), _full_spec((1, _VW))],
        out_specs=[_row_spec(_LB, _D), _row_spec(_LB, _D),
                   _row_spec(_LB, _VW)],
        out_shape=[s, s, sv],
    )(h, wq, bq, wk, bk, wv, bv)


def _mega(q, k, v, h, wo, bo, g1, be1, w1, b1, w2, b2, g2, be2):
    return pl.pallas_call(
        _mega_body,
        grid=(_L // _LQB,),
        in_specs=[_row_spec(_LQB, _D), _full_spec((_L, _D)),
                  _full_spec((_L, _VW)), _row_spec(_LQB, _D),
                  _full_spec((_D, _D)), _full_spec((1, _D)),
                  _full_spec((1, _D)), _full_spec((1, _D)),
                  _full_spec((_D, _FF)), _full_spec((1, _FF)),
                  _full_spec((_FF, _D)), _full_spec((1, _D)),
                  _full_spec((1, _D)), _full_spec((1, _D))],
        out_specs=_row_spec(_LQB, _D),
        out_shape=jax.ShapeDtypeStruct((_L, _D), _F32),
    )(q, k, v, h, wo, bo, g1, be1, w1, b1, w2, b2, g2, be2)


def _outproj(h, wout, bout):
    return pl.pallas_call(
        _outproj_body,
        grid=(_L // _LB,),
        in_specs=[_row_spec(_LB, _D), _full_spec((_D, _IN)),
                  _full_spec((1, _IN))],
        out_specs=_row_spec(_LB, _IN),
        out_shape=jax.ShapeDtypeStruct((_L, _IN), _F32),
    )(h, wout, bout)


def kernel(x_t, t_embed, Win, b_in, Wq, bq, Wk, bk, Wv, bv, Wo, bo,
           W1, b1, W2, b2, g1, be1, g2, be2, Wout, bout):
    x = x_t.reshape(_L, _IN)
    pe = _pos_enc()
    h = _inproj(x, Win.astype(_BF), b_in.reshape(1, _D), pe,
                t_embed.reshape(1, _D))
    _C = _SCALE * (1.0 / math.log(2.0))
    Wqb = (Wq * _C).astype(_BF)
    bqb = (bq * _C).astype(_BF)
    Wkb, Wob = Wk.astype(_BF), Wo.astype(_BF)
    bkb = bk.astype(_BF)
    W1b, W2b = W1.astype(_BF), W2.astype(_BF)
    b1b = b1.astype(_BF)
    Wv_aug = jnp.pad(Wv.reshape(_LAYERS, _D, _H, _DH),
                     ((0, 0), (0, 0), (0, 0), (0, 128 - _DH))
                     ).reshape(_LAYERS, _D, _VW).astype(_BF)
    bv_aug = jnp.concatenate(
        [bv.reshape(_LAYERS, _H, _DH),
         jnp.ones((_LAYERS, _H, 1), _F32),
         jnp.zeros((_LAYERS, _H, 128 - _DH - 1), _F32)],
        axis=-1).reshape(_LAYERS, _VW).astype(_BF)
    for i in range(_LAYERS):
        q, k, v = _qkv(h, Wqb[i], bqb[i].reshape(1, _D), Wkb[i],
                       bkb[i].reshape(1, _D), Wv_aug[i],
                       bv_aug[i].reshape(1, _VW))
        h = _mega(q, k, v, h, Wob[i], bo[i].reshape(1, _D),
                  g1[i].reshape(1, _D), be1[i].reshape(1, _D),
                  W1b[i], b1b[i].reshape(1, _FF), W2b[i],
                  b2[i].reshape(1, _D), g2[i].reshape(1, _D),
                  be2[i].reshape(1, _D))
    out = _outproj(h, Wout.astype(_BF), bout.reshape(1, _IN))
    return out.reshape(_B, _L, _IN)
